# Initial kernel scaffold; baseline (speedup 1.0000x reference)
import functools

import jax
import jax.numpy as jnp
from jax.experimental import pallas as pl
from jax.experimental.pallas import tpu as pltpu
from jax.experimental.pallas import tpu_sc as plsc

_W = 1024


def _gather_fn(table, idx_flat, n_rows, emb_dim):
    mesh = plsc.VectorSubcoreMesh(core_axis_name="c", subcore_axis_name="s")

    @functools.partial(
        pl.kernel,
        out_type=jax.ShapeDtypeStruct((n_rows, emb_dim), table.dtype),
        mesh=mesh,
    )
    def gather_kernel(table_hbm, idx_hbm, out_hbm):
        def body(idx_v, out_v):
            pltpu.sync_copy(table_hbm.at[idx_v.at[0]], out_v)

        pltpu.emit_pipeline(
            body,
            grid=(n_rows // _W,),
            in_specs=[pl.BlockSpec((1, _W), lambda i: (0, i))],
            out_specs=[pl.BlockSpec((_W, emb_dim), lambda i: (i, 0))],
            core_axis_name=("c", "s"),
            dimension_semantics=(pltpu.PARALLEL,),
        )(idx_hbm, out_hbm)

    return gather_kernel(table, idx_flat)


def kernel(x, embedding):
    batch, fields = x.shape
    vocab, emb_dim = embedding.shape
    n_rows = batch * fields
    idx_flat = x.reshape(1, n_rows)
    out = _gather_fn(embedding, idx_flat, n_rows, emb_dim)
    return out.reshape(batch, fields, emb_dim)


# speedup vs baseline: 1.3673x; 1.3673x over previous
import functools

import jax
import jax.numpy as jnp
from jax import lax
from jax.experimental import pallas as pl
from jax.experimental.pallas import tpu as pltpu
from jax.experimental.pallas import tpu_sc as plsc

_NC = 2
_NS = 16
_NW = _NC * _NS
_C = 128


def _gather_fn(table, idx_flat, n_rows, emb_dim):
    mesh = plsc.VectorSubcoreMesh(core_axis_name="c", subcore_axis_name="s")
    n_per_w = n_rows // _NW
    steps = n_per_w // _C

    @functools.partial(
        pl.kernel,
        out_type=jax.ShapeDtypeStruct((n_rows, emb_dim), table.dtype),
        mesh=mesh,
        compiler_params=pltpu.CompilerParams(use_tc_tiling_on_sc=False),
        scratch_types=[
            pltpu.VMEM((_C,), jnp.int32),
            pltpu.VMEM((_C, emb_dim), table.dtype),
            pltpu.SemaphoreType.DMA,
        ],
    )
    def gather_kernel(table_hbm, idx_hbm, out_hbm, idx_v, rows_v, sem):
        wid = lax.axis_index("s") * _NC + lax.axis_index("c")
        base = wid * n_per_w

        @pl.loop(0, steps)
        def _(i):
            off = base + i * _C
            pltpu.sync_copy(idx_hbm.at[pl.ds(off, _C)], idx_v)
            pltpu.async_copy(table_hbm.at[idx_v], rows_v, sem).wait()
            pltpu.sync_copy(rows_v, out_hbm.at[pl.ds(off, _C)])

    return gather_kernel(table, idx_flat)


def kernel(x, embedding):
    batch, fields = x.shape
    vocab, emb_dim = embedding.shape
    n_rows = batch * fields
    idx_flat = x.reshape(n_rows)
    out = _gather_fn(embedding, idx_flat, n_rows, emb_dim)
    return out.reshape(batch, fields, emb_dim)


# speedup vs baseline: 1.5547x; 1.1370x over previous
import functools

import jax
import jax.numpy as jnp
from jax import lax
from jax.experimental import pallas as pl
from jax.experimental.pallas import tpu as pltpu
from jax.experimental.pallas import tpu_sc as plsc

_NC = 2
_NS = 16
_NW = _NC * _NS
_R = 64


def _gather_fn(table, x):
    batch, fields = x.shape
    vocab, emb_dim = table.shape
    mesh = plsc.VectorSubcoreMesh(core_axis_name="c", subcore_axis_name="s")
    rows_per_w = batch // _NW
    steps = rows_per_w // _R

    @functools.partial(
        pl.kernel,
        out_type=jax.ShapeDtypeStruct((batch, fields, emb_dim), table.dtype),
        mesh=mesh,
        compiler_params=pltpu.CompilerParams(use_tc_tiling_on_sc=False),
        scratch_types=[
            pltpu.VMEM((_R, fields), jnp.int32),
            pltpu.VMEM((_R, fields, emb_dim), table.dtype),
            pltpu.SemaphoreType.DMA,
        ],
    )
    def gather_kernel(table_hbm, idx_hbm, out_hbm, idx_v, rows_v, sem):
        wid = lax.axis_index("s") * _NC + lax.axis_index("c")
        base = wid * rows_per_w

        @pl.loop(0, steps)
        def _(i):
            row0 = base + i * _R
            pltpu.sync_copy(idx_hbm.at[pl.ds(row0, _R)], idx_v)

            @pl.loop(0, _R)
            def _(r):
                pltpu.async_copy(table_hbm.at[idx_v.at[r]], rows_v.at[r], sem)

            pltpu.make_async_copy(
                out_hbm.at[pl.ds(row0, _R)], rows_v, sem
            ).wait()
            pltpu.sync_copy(rows_v, out_hbm.at[pl.ds(row0, _R)])

    return gather_kernel(table, x)


def kernel(x, embedding):
    return _gather_fn(embedding, x)
